# baseline (speedup 1.0000x reference)
import functools

import jax
import jax.numpy as jnp
from jax import lax
from jax.experimental import pallas as pl
from jax.experimental.pallas import tpu as pltpu
from jax.experimental.pallas import tpu_sc as plsc

E = 320000
D = 128
NW = 32
E_PER_W = E // NW
V = 10000
B = 200
BP = 208
NCHUNK = E_PER_W // B
NPAIR = NCHUNK // 2
NGROUP = BP // 16

_mesh = plsc.VectorSubcoreMesh(core_axis_name="c", subcore_axis_name="s")

_SHUFFLE_DNUMS = lax.GatherDimensionNumbers(
    offset_dims=(), collapsed_slice_dims=(0,), start_index_map=(0,))


def _lane_shuffle(x, idx):
    return lax.gather(x, idx[:, None], _SHUFFLE_DNUMS, (1,),
                      mode=lax.GatherScatterMode.PROMISE_IN_BOUNDS)


def _unpack_bf16_pair(x_i32):
    lo = lax.bitcast_convert_type(lax.shift_left(x_i32, 16), jnp.float32)
    hi = lax.bitcast_convert_type(
        lax.bitwise_and(x_i32, jnp.int32(-65536)), jnp.float32)
    return lo, hi


@functools.partial(
    pl.kernel,
    mesh=_mesh,
    out_type=jax.ShapeDtypeStruct((E,), jnp.float32),
    scratch_types=[
        pltpu.VMEM((2 * B,), jnp.int32),
        pltpu.VMEM((2 * B,), jnp.int32),
        pltpu.VMEM((2 * BP, D), jnp.float32),
        pltpu.VMEM((2 * BP, D), jnp.float32),
        pltpu.VMEM((BP,), jnp.float32),
        pltpu.VMEM((BP,), jnp.float32),
        pltpu.SemaphoreType.DMA((2,)),
        pltpu.SemaphoreType.DMA((2,)),
        pltpu.SemaphoreType.DMA((2,)),
    ],
)
def _decoder_sc(z_hbm, src_hbm, dst_hbm, out_hbm,
                idx0_v, idx1_v, rows0_v, rows1_v, o0_v, o1_v,
                isem, rsem, osem):
    wid = lax.axis_index("s") * 2 + lax.axis_index("c")
    base = wid * E_PER_W
    lane = lax.iota(jnp.int32, 16)
    rows_v = (rows0_v, rows1_v)
    idxs_v = (idx0_v, idx1_v)
    os_v = (o0_v, o1_v)

    def fetch_idx(c, b):
        off = base + c * B
        pltpu.async_copy(src_hbm.at[pl.ds(off, B)],
                         idxs_v[b].at[pl.ds(0, B)], isem.at[b])
        pltpu.async_copy(dst_hbm.at[pl.ds(off, B)],
                         idxs_v[b].at[pl.ds(B, B)], isem.at[b])

    def wait_idx(b):
        pltpu.make_async_copy(src_hbm.at[pl.ds(0, B)],
                              idxs_v[b].at[pl.ds(0, B)], isem.at[b]).wait()
        pltpu.make_async_copy(dst_hbm.at[pl.ds(0, B)],
                              idxs_v[b].at[pl.ds(B, B)], isem.at[b]).wait()

    _SPLITS = ((0, 104), (104, 96), (200, 104), (304, 96))

    def start_gather(b):
        for off, n in _SPLITS:
            pltpu.async_copy(z_hbm.at[idxs_v[b].at[pl.ds(off, n)]],
                             rows_v[b].at[pl.ds(off, n)], rsem.at[b])

    def wait_gather(b):
        for off, n in _SPLITS:
            pltpu.make_async_copy(z_hbm.at[idxs_v[b].at[pl.ds(off, n)]],
                                  rows_v[b].at[pl.ds(off, n)],
                                  rsem.at[b]).wait()

    def compute(c, b):
        rv = rows_v[b]

        def group_body(g, carry):
            e0 = g * 16
            tot = jnp.zeros((16,), jnp.float32)
            for e16 in range(16):
                e = e0 + e16
                prods = [rv[e, pl.ds(k * 16, 16)] * rv[B + e, pl.ds(k * 16, 16)]
                         for k in range(D // 16)]
                while len(prods) > 1:
                    prods = [a + c for a, c in zip(prods[::2], prods[1::2])]
                acc = prods[0]
                for shift in (8, 4, 2, 1):
                    acc = acc + _lane_shuffle(acc, lane ^ shift)
                tot = jnp.where(lane == e16, acc, tot)
            os_v[b][pl.ds(e0, 16)] = tot
            return carry

        lax.fori_loop(0, NGROUP, group_body, 0)
        pltpu.async_copy(os_v[b].at[pl.ds(0, B)],
                         out_hbm.at[pl.ds(base + c * B, B)], osem.at[b])

    def wait_out(b):
        pltpu.make_async_copy(os_v[b].at[pl.ds(0, B)],
                              out_hbm.at[pl.ds(0, B)], osem.at[b]).wait()

    fetch_idx(0, 0)
    wait_idx(0)
    start_gather(0)
    fetch_idx(1, 1)

    def pair_body(g, carry):
        c0 = 2 * g
        wait_gather(0)

        @pl.when(g < NPAIR - 1)
        def _prefetch_even():
            fetch_idx(c0 + 2, 0)

        wait_idx(1)
        start_gather(1)

        @pl.when(g >= 1)
        def _drain_out0():
            wait_out(0)

        compute(c0, 0)
        wait_gather(1)

        @pl.when(g < NPAIR - 1)
        def _prefetch_odd():
            fetch_idx(c0 + 3, 1)

        @pl.when(g < NPAIR - 1)
        def _gather_even():
            wait_idx(0)
            start_gather(0)

        @pl.when(g >= 1)
        def _drain_out1():
            wait_out(1)

        compute(c0 + 1, 1)
        return carry

    lax.fori_loop(0, NPAIR, pair_body, 0)
    wait_out(0)
    wait_out(1)


def kernel(z, edge_index):
    ei = edge_index.astype(jnp.int32)
    return _decoder_sc(z, ei[0], ei[1])


# speedup vs baseline: 2.1909x; 2.1909x over previous
import functools

import jax
import jax.numpy as jnp
from jax import lax
from jax.experimental import pallas as pl
from jax.experimental.pallas import tpu as pltpu
from jax.experimental.pallas import tpu_sc as plsc

E = 320000
D = 128
V = 10000
NW = 32
E_PER_W = E // NW
B = 200
BP = 208
NCHUNK = E_PER_W // B
NPAIR = NCHUNK // 2
NGROUP = BP // 16
IDX_PER_W = NCHUNK * 2 * B

_mesh = plsc.VectorSubcoreMesh(core_axis_name="c", subcore_axis_name="s")

_SHUFFLE_DNUMS = lax.GatherDimensionNumbers(
    offset_dims=(), collapsed_slice_dims=(0,), start_index_map=(0,))


def _lane_shuffle(x, idx):
    return lax.gather(x, idx[:, None], _SHUFFLE_DNUMS, (1,),
                      mode=lax.GatherScatterMode.PROMISE_IN_BOUNDS)


@functools.partial(
    pl.kernel,
    mesh=_mesh,
    out_type=jax.ShapeDtypeStruct((E,), jnp.float32),
    scratch_types=[
        pltpu.VMEM((IDX_PER_W,), jnp.int32),
        pltpu.VMEM((B + BP, D), jnp.float32),
        pltpu.VMEM((B + BP, D), jnp.float32),
        pltpu.VMEM((BP,), jnp.float32),
        pltpu.VMEM((BP,), jnp.float32),
        pltpu.SemaphoreType.DMA((2,)),
        pltpu.SemaphoreType.DMA((2,)),
    ],
)
def _decoder_sc(z_hbm, comb_hbm, out_hbm,
                cidx_v, rows0_v, rows1_v, o0_v, o1_v, rsem, osem):
    wid = lax.axis_index("s") * 2 + lax.axis_index("c")
    base = wid * E_PER_W
    lane = lax.iota(jnp.int32, 16)
    rows_v = (rows0_v, rows1_v)
    os_v = (o0_v, o1_v)

    def start_gather(c, b):
        pltpu.async_copy(z_hbm.at[cidx_v.at[pl.ds(c * 2 * B, 2 * B)]],
                         rows_v[b].at[pl.ds(0, 2 * B)], rsem.at[b])

    def wait_gather(b):
        pltpu.make_async_copy(z_hbm.at[cidx_v.at[pl.ds(0, 2 * B)]],
                              rows_v[b].at[pl.ds(0, 2 * B)],
                              rsem.at[b]).wait()

    def compute(c, b):
        rv = rows_v[b]

        def group_body(g, carry):
            e0 = g * 16

            def half_body(h, tot):
                eh = e0 + h * 8
                for e8 in range(8):
                    e = eh + e8
                    acc = rv[e, pl.ds(0, 16)] * rv[B + e, pl.ds(0, 16)]
                    for k in range(1, D // 16):
                        acc += (rv[e, pl.ds(k * 16, 16)]
                                * rv[B + e, pl.ds(k * 16, 16)])
                    for shift in (8, 4, 2, 1):
                        acc = acc + _lane_shuffle(acc, lane ^ shift)
                    tot = jnp.where(lane == h * 8 + e8, acc, tot)
                return tot

            tot = lax.fori_loop(0, 2, half_body, jnp.zeros((16,), jnp.float32))
            os_v[b][pl.ds(e0, 16)] = tot
            return carry

        lax.fori_loop(0, NGROUP, group_body, 0)
        pltpu.async_copy(os_v[b].at[pl.ds(0, B)],
                         out_hbm.at[pl.ds(base + c * B, B)], osem.at[b])

    def wait_out(b):
        pltpu.make_async_copy(os_v[b].at[pl.ds(0, B)],
                              out_hbm.at[pl.ds(0, B)], osem.at[b]).wait()

    pltpu.sync_copy(comb_hbm.at[pl.ds(wid * IDX_PER_W, IDX_PER_W)], cidx_v)
    start_gather(0, 0)
    start_gather(1, 1)

    def pair_body(g, carry):
        c0 = 2 * g
        wait_gather(0)

        @pl.when(g >= 1)
        def _drain_out0():
            wait_out(0)

        compute(c0, 0)

        @pl.when(g < NPAIR - 1)
        def _gather_even():
            start_gather(c0 + 2, 0)

        wait_gather(1)

        @pl.when(g >= 1)
        def _drain_out1():
            wait_out(1)

        compute(c0 + 1, 1)

        @pl.when(g < NPAIR - 1)
        def _gather_odd():
            start_gather(c0 + 3, 1)

        return carry

    lax.fori_loop(0, NPAIR, pair_body, 0)
    wait_out(0)
    wait_out(1)


def kernel(z, edge_index):
    ei = edge_index.astype(jnp.int32)
    comb = (ei.reshape(2, NW, NCHUNK, B)
            .transpose(1, 2, 0, 3)
            .reshape(NW * IDX_PER_W))
    return _decoder_sc(z, comb)


# speedup vs baseline: 2.2612x; 1.0321x over previous
import functools

import jax
import jax.numpy as jnp
from jax import lax
from jax.experimental import pallas as pl
from jax.experimental.pallas import tpu as pltpu
from jax.experimental.pallas import tpu_sc as plsc

E = 320000
D = 128
V = 10000
NW = 32
E_PER_W = E // NW
B = 200
BP = 208
NCHUNK = E_PER_W // B
NPAIR = NCHUNK // 2
NGROUP = BP // 16
IDX_PER_W = NCHUNK * 2 * B

_mesh = plsc.VectorSubcoreMesh(core_axis_name="c", subcore_axis_name="s")

_SHUFFLE_DNUMS = lax.GatherDimensionNumbers(
    offset_dims=(), collapsed_slice_dims=(0,), start_index_map=(0,))


def _lane_shuffle(x, idx):
    return lax.gather(x, idx[:, None], _SHUFFLE_DNUMS, (1,),
                      mode=lax.GatherScatterMode.PROMISE_IN_BOUNDS)


def _unpack_bf16_pair(x_i32):
    lo = lax.bitcast_convert_type(lax.shift_left(x_i32, 16), jnp.float32)
    hi = lax.bitcast_convert_type(
        lax.bitwise_and(x_i32, jnp.int32(-65536)), jnp.float32)
    return lo, hi


@functools.partial(
    pl.kernel,
    mesh=_mesh,
    compiler_params=pltpu.CompilerParams(use_tc_tiling_on_sc=False),
    out_type=jax.ShapeDtypeStruct((E,), jnp.float32),
    scratch_types=[
        pltpu.VMEM((IDX_PER_W,), jnp.int32),
        pltpu.VMEM((B + BP, D // 2), jnp.int32),
        pltpu.VMEM((B + BP, D // 2), jnp.int32),
        pltpu.VMEM((BP,), jnp.float32),
        pltpu.VMEM((BP,), jnp.float32),
        pltpu.SemaphoreType.DMA((2,)),
        pltpu.SemaphoreType.DMA((2,)),
    ],
)
def _decoder_sc(z_hbm, comb_hbm, out_hbm,
                cidx_v, rows0_v, rows1_v, o0_v, o1_v, rsem, osem):
    wid = lax.axis_index("s") * 2 + lax.axis_index("c")
    base = wid * E_PER_W
    lane = lax.iota(jnp.int32, 16)
    rows_v = (rows0_v, rows1_v)
    os_v = (o0_v, o1_v)

    def start_gather(c, b):
        pltpu.async_copy(z_hbm.at[cidx_v.at[pl.ds(c * 2 * B, 2 * B)]],
                         rows_v[b].at[pl.ds(0, 2 * B)], rsem.at[b])

    def wait_gather(b):
        pltpu.make_async_copy(z_hbm.at[cidx_v.at[pl.ds(0, 2 * B)]],
                              rows_v[b].at[pl.ds(0, 2 * B)],
                              rsem.at[b]).wait()

    def compute(c, b):
        rv = rows_v[b]

        def group_body(g, carry):
            e0 = g * 16

            def half_body(h, tot):
                eh = e0 + h * 8
                for e8 in range(8):
                    e = eh + e8
                    acc = jnp.zeros((16,), jnp.float32)
                    for k in range(D // 32):
                        sa, sb = _unpack_bf16_pair(rv[e, pl.ds(k * 16, 16)])
                        ta, tb = _unpack_bf16_pair(rv[B + e, pl.ds(k * 16, 16)])
                        acc += sa * ta
                        acc += sb * tb
                    for shift in (8, 4, 2, 1):
                        acc = acc + _lane_shuffle(acc, lane ^ shift)
                    tot = jnp.where(lane == h * 8 + e8, acc, tot)
                return tot

            tot = lax.fori_loop(0, 2, half_body, jnp.zeros((16,), jnp.float32))
            os_v[b][pl.ds(e0, 16)] = tot
            return carry

        lax.fori_loop(0, NGROUP, group_body, 0)
        pltpu.async_copy(os_v[b].at[pl.ds(0, B)],
                         out_hbm.at[pl.ds(base + c * B, B)], osem.at[b])

    def wait_out(b):
        pltpu.make_async_copy(os_v[b].at[pl.ds(0, B)],
                              out_hbm.at[pl.ds(0, B)], osem.at[b]).wait()

    pltpu.sync_copy(comb_hbm.at[pl.ds(wid * IDX_PER_W, IDX_PER_W)], cidx_v)
    start_gather(0, 0)
    start_gather(1, 1)

    def pair_body(g, carry):
        c0 = 2 * g
        wait_gather(0)

        @pl.when(g >= 1)
        def _drain_out0():
            wait_out(0)

        compute(c0, 0)

        @pl.when(g < NPAIR - 1)
        def _gather_even():
            start_gather(c0 + 2, 0)

        wait_gather(1)

        @pl.when(g >= 1)
        def _drain_out1():
            wait_out(1)

        compute(c0 + 1, 1)

        @pl.when(g < NPAIR - 1)
        def _gather_odd():
            start_gather(c0 + 3, 1)

        return carry

    lax.fori_loop(0, NPAIR, pair_body, 0)
    wait_out(0)
    wait_out(1)


def kernel(z, edge_index):
    ei = edge_index.astype(jnp.int32)
    comb = (ei.reshape(2, NW, NCHUNK, B)
            .transpose(1, 2, 0, 3)
            .reshape(NW * IDX_PER_W))
    zp = lax.bitcast_convert_type(
        z.astype(jnp.bfloat16).reshape(V, D // 2, 2), jnp.int32)
    return _decoder_sc(zp, comb)
